# baseline (speedup 1.0000x reference)
import functools

import jax
import jax.numpy as jnp
from jax import lax
from jax.experimental import pallas as pl
from jax.experimental.pallas import tpu as pltpu

ALPHA = 0.25
GAMMA = 2.0
COST_CLASS = 2.0
COST_BBOX = 5.0
COST_GIOU = 2.0
REF_DISTANCE = 0.1
INF = 1e18
BIG = 1 << 30


def _cost_kernel(nq, nt, ncls, npad, logits_ref, dx_ref, pb_ref, pbT_ref,
                 tlT_ref, ids_ref, tbT_ref, tbb_ref, c_ref, w_ref):
    R = npad // 128

    lg = logits_ref[0]
    prob = jax.nn.sigmoid(lg)
    neg = (1.0 - ALPHA) * (prob * prob) * -jnp.log(1.0 - prob + 1e-08)
    pos = ALPHA * ((1.0 - prob) * (1.0 - prob)) * -jnp.log(prob + 1e-08)
    diff = pos - neg

    ids = ids_ref[...]
    nall = ids.shape[1]
    e_full = (lax.broadcasted_iota(jnp.int32, (ncls, nall), 0) == ids
              ).astype(jnp.float32)
    cc = jnp.dot(diff, e_full, preferred_element_type=jnp.float32)

    ob = pb_ref[0]
    tbT = tbT_ref[...]
    cb = (jnp.abs(ob[:, 0:1] - tbT[0:1, :]) + jnp.abs(ob[:, 1:2] - tbT[1:2, :])
          + jnp.abs(ob[:, 2:3] - tbT[2:3, :]) + jnp.abs(ob[:, 3:4] - tbT[3:4, :]))

    ox0 = ob[:, 0:1] - 0.5 * ob[:, 2:3]
    oy0 = ob[:, 1:2] - 0.5 * ob[:, 3:4]
    ox1 = ob[:, 0:1] + 0.5 * ob[:, 2:3]
    oy1 = ob[:, 1:2] + 0.5 * ob[:, 3:4]
    tx0 = tbT[0:1, :] - 0.5 * tbT[2:3, :]
    ty0 = tbT[1:2, :] - 0.5 * tbT[3:4, :]
    tx1 = tbT[0:1, :] + 0.5 * tbT[2:3, :]
    ty1 = tbT[1:2, :] + 0.5 * tbT[3:4, :]
    area1 = (ox1 - ox0) * (oy1 - oy0)
    area2 = (tx1 - tx0) * (ty1 - ty0)
    iw = jnp.maximum(jnp.minimum(ox1, tx1) - jnp.maximum(ox0, tx0), 0.0)
    ih = jnp.maximum(jnp.minimum(oy1, ty1) - jnp.maximum(oy0, ty0), 0.0)
    inter = iw * ih
    union = area1 + area2 - inter
    iou = inter / union
    ew = jnp.maximum(jnp.maximum(ox1, tx1) - jnp.minimum(ox0, tx0), 0.0)
    eh = jnp.maximum(jnp.maximum(oy1, ty1) - jnp.minimum(oy0, ty0), 0.0)
    earea = ew * eh
    giou = iou - (earea - union) / earea
    c_ref[0] = COST_BBOX * cb + COST_CLASS * cc + COST_GIOU * (-giou)

    idsb = tlT_ref[0]
    eb = (lax.broadcasted_iota(jnp.int32, (nt, ncls), 1) == idsb
          ).astype(jnp.float32)
    ccT = lax.dot_general(eb, dx_ref[0], (((1,), (1,)), ((), ())),
                          preferred_element_type=jnp.float32,
                          precision=lax.Precision.HIGHEST)

    obT = pbT_ref[0]
    tbb = tbb_ref[0]
    cbT = (jnp.abs(tbb[:, 0:1] - obT[0:1, :]) + jnp.abs(tbb[:, 1:2] - obT[1:2, :])
           + jnp.abs(tbb[:, 2:3] - obT[2:3, :]) + jnp.abs(tbb[:, 3:4] - obT[3:4, :]))

    qx0 = obT[0:1, :] - 0.5 * obT[2:3, :]
    qy0 = obT[1:2, :] - 0.5 * obT[3:4, :]
    qx1 = obT[0:1, :] + 0.5 * obT[2:3, :]
    qy1 = obT[1:2, :] + 0.5 * obT[3:4, :]
    gx0 = tbb[:, 0:1] - 0.5 * tbb[:, 2:3]
    gy0 = tbb[:, 1:2] - 0.5 * tbb[:, 3:4]
    gx1 = tbb[:, 0:1] + 0.5 * tbb[:, 2:3]
    gy1 = tbb[:, 1:2] + 0.5 * tbb[:, 3:4]
    areaq = (qx1 - qx0) * (qy1 - qy0)
    areag = (gx1 - gx0) * (gy1 - gy0)
    iwT = jnp.maximum(jnp.minimum(qx1, gx1) - jnp.maximum(qx0, gx0), 0.0)
    ihT = jnp.maximum(jnp.minimum(qy1, gy1) - jnp.maximum(qy0, gy0), 0.0)
    interT = iwT * ihT
    unionT = areaq + areag - interT
    iouT = interT / unionT
    ewT = jnp.maximum(jnp.maximum(qx1, gx1) - jnp.minimum(qx0, gx0), 0.0)
    ehT = jnp.maximum(jnp.maximum(qy1, gy1) - jnp.minimum(qy0, gy0), 0.0)
    eareaT = ewT * ehT
    giouT = iouT - (eareaT - unionT) / eareaT
    subT = COST_BBOX * cbT + COST_CLASS * ccT + COST_GIOU * (-giouT)

    subT_pad = jnp.concatenate(
        [subT, jnp.zeros((nt, npad - nq), jnp.float32)], axis=1)
    for k in range(R):
        w_ref[:, 0, k, :] = subT_pad[:, k * 128:(k + 1) * 128]


def _assign_kernel(bs, nq, nt, npad, w_ref, tbb_ref, rpT_ref, match_ref):
    R = npad // 128
    nrow_pad = 32

    fi3 = (lax.broadcasted_iota(jnp.int32, (1, R, 128), 1) * 128
           + lax.broadcasted_iota(jnp.int32, (1, R, 128), 2))
    pad_used = jnp.broadcast_to((fi3 >= nq).astype(jnp.float32), (bs, R, 128))
    riota = lax.broadcasted_iota(jnp.int32, (1, nrow_pad), 1)

    u0 = jnp.zeros((bs, nrow_pad), jnp.float32)
    v0 = jnp.zeros((bs, R, 128), jnp.float32)
    p0 = jnp.full((bs, R, 128), -1, jnp.int32)
    way0 = jnp.full((bs, R, 128), -1, jnp.int32)

    def row_body(i, carry):
        u, v, p, way = carry
        packed = fi3 * 32 + (p + 1)

        def relax(rows, ucur, act, j0, minv, usedc, usedr, way, u, v):
            actb = act > 0
            act8 = actb[:, :, 0]
            cur = rows - ucur[:, :, None] - v
            freeb = usedc < 0.5
            better = actb & freeb & (cur < minv)
            minv = jnp.where(better, cur, minv)
            way = jnp.where(better, j0, way)
            masked = jnp.where(freeb, minv, INF)
            delta = jnp.min(jnp.min(masked, axis=1, keepdims=True),
                            axis=2, keepdims=True)
            key = jnp.where(actb & (masked == delta), packed, jnp.int32(BIG))
            kmin = jnp.min(jnp.min(key, axis=1, keepdims=True),
                           axis=2, keepdims=True)
            j1 = kmin // 32
            rnew = (kmin & 31) - 1
            adelta = jnp.where(act8, delta[:, :, 0], 0.0)
            u = u + adelta * usedr
            v = v - adelta[:, :, None] * usedc
            minv = jnp.where(freeb, minv - adelta[:, :, None], minv)
            selj = fi3 == j1
            usedc = jnp.where(selj, 1.0, usedc)
            usedr = jnp.where((riota == rnew[:, :, 0]) & act8, 1.0, usedr)
            j0 = jnp.where(actb, j1, j0)
            act = jnp.where(actb & (rnew >= 0), 1, 0)
            rnext = jnp.maximum(rnew, 0)
            return (act, rnext, j0, minv, usedc, usedr, way, u, v)

        rows1 = w_ref[i]
        ucur1 = jnp.sum(u * (riota == i).astype(jnp.float32),
                        axis=1, keepdims=True)
        st1 = relax(rows1, ucur1,
                    jnp.ones((bs, 1, 1), jnp.int32),
                    jnp.full((bs, 1, 1), -1, jnp.int32),
                    jnp.full((bs, R, 128), INF, jnp.float32),
                    pad_used,
                    jnp.broadcast_to((riota == i).astype(jnp.float32),
                                     (bs, nrow_pad)),
                    way, u, v)

        def s_cond(st):
            return jnp.max(st[0]) > 0

        def s_body(st):
            act, r, j0, minv, usedc, usedr, way, u, v = st
            rows = []
            for im in range(bs):
                rows.append(w_ref[r[im, 0, 0], im])
            rows = jnp.stack(rows, axis=0)
            ucur = jnp.sum(u * (riota == r[:, :, 0]).astype(jnp.float32),
                           axis=1, keepdims=True)
            return relax(rows, ucur, act, j0, minv, usedc, usedr, way, u, v)

        (_, _, jfin, _, _, _, way, u, v) = lax.while_loop(s_cond, s_body, st1)

        def aug_step(act, j, p):
            actb = act > 0
            selj = actb & (fi3 == j)
            pj = jnp.min(jnp.min(jnp.where(selj, way, jnp.int32(BIG)),
                                 axis=1, keepdims=True),
                         axis=2, keepdims=True)
            pj = jnp.where(actb, pj, jnp.int32(-2))
            pval = jnp.sum(jnp.sum(jnp.where(fi3 == pj, p, 0),
                                   axis=1, keepdims=True),
                           axis=2, keepdims=True)
            newp = jnp.where(pj < 0, i, pval)
            p = jnp.where(selj, newp, p)
            act = jnp.where(actb & (pj >= 0), 1, 0)
            return (act, pj, p)

        act_a, j_a, p = aug_step(jnp.ones((bs, 1, 1), jnp.int32), jfin, p)

        def a_cond(st):
            return jnp.max(st[0]) > 0

        def a_body(st):
            return aug_step(*st)

        _, _, p = lax.while_loop(a_cond, a_body, (act_a, j_a, p))
        return (u, v, p, way)

    _, _, p, _ = lax.fori_loop(0, nt, row_body, (u0, v0, p0, way0))

    tbb = tbb_ref[...]
    gcx = tbb[:, :, 0:1]
    gcy = tbb[:, :, 1:2]
    tiota = lax.broadcasted_iota(jnp.int32, (bs, nt, 128), 1)
    mind_rows = []
    midx_rows = []
    for k in range(R):
        rx = rpT_ref[:, 0:1, k * 128:(k + 1) * 128]
        ry = rpT_ref[:, 1:2, k * 128:(k + 1) * 128]
        dxk = rx - gcx
        dyk = ry - gcy
        dk = jnp.sqrt(dxk * dxk + dyk * dyk)
        mk = jnp.min(dk, axis=1, keepdims=True)
        ik = jnp.min(jnp.where(dk == mk, tiota, jnp.int32(BIG)), axis=1, keepdims=True)
        mind_rows.append(mk)
        midx_rows.append(ik)
    min_d = jnp.concatenate(mind_rows, axis=1)
    min_idx = jnp.concatenate(midx_rows, axis=1)
    sel = (p < 0) & (min_d <= REF_DISTANCE)
    match_ref[...] = jnp.where(sel, min_idx, p)


def kernel(pred_logits, pred_boxes, tgt_labels, tgt_boxes, reference_points):
    bs, nq, ncls = pred_logits.shape
    nt = tgt_boxes.shape[1]
    npad = ((nq + 1023) // 1024) * 1024
    R = npad // 128

    out_prob = jax.nn.sigmoid(pred_logits.reshape(bs * nq, ncls))
    neg_c = (1 - ALPHA) * out_prob ** GAMMA * -jnp.log(1 - out_prob + 1e-08)
    pos_c = ALPHA * (1 - out_prob) ** GAMMA * -jnp.log(out_prob + 1e-08)
    dx = (pos_c - neg_c).reshape(bs, nq, ncls)

    ids_full = tgt_labels.reshape(1, bs * nt).astype(jnp.int32)
    tlT = tgt_labels.astype(jnp.int32).reshape(bs, nt, 1)
    tbT_full = tgt_boxes.reshape(bs * nt, 4).T
    pbT = jnp.swapaxes(pred_boxes, 1, 2)
    rpT = jnp.swapaxes(reference_points, 1, 2)
    rpT_pad = jnp.pad(rpT, ((0, 0), (0, 0), (0, npad - nq)),
                      constant_values=1e6)

    body_a = functools.partial(_cost_kernel, nq, nt, ncls, npad)
    C, W = pl.pallas_call(
        body_a,
        grid=(bs,),
        in_specs=[
            pl.BlockSpec((1, nq, ncls), lambda b: (b, 0, 0)),
            pl.BlockSpec((1, nq, ncls), lambda b: (b, 0, 0)),
            pl.BlockSpec((1, nq, 4), lambda b: (b, 0, 0)),
            pl.BlockSpec((1, 4, nq), lambda b: (b, 0, 0)),
            pl.BlockSpec((1, nt, 1), lambda b: (b, 0, 0)),
            pl.BlockSpec((1, bs * nt), lambda b: (0, 0)),
            pl.BlockSpec((4, bs * nt), lambda b: (0, 0)),
            pl.BlockSpec((1, nt, 4), lambda b: (b, 0, 0)),
        ],
        out_specs=[
            pl.BlockSpec((1, nq, bs * nt), lambda b: (b, 0, 0)),
            pl.BlockSpec((nt, 1, R, 128), lambda b: (0, b, 0, 0)),
        ],
        out_shape=[
            jax.ShapeDtypeStruct((bs, nq, bs * nt), jnp.float32),
            jax.ShapeDtypeStruct((nt, bs, R, 128), jnp.float32),
        ],
        compiler_params=pltpu.CompilerParams(
            dimension_semantics=("arbitrary",)),
    )(pred_logits, dx, pred_boxes, pbT, tlT, ids_full, tbT_full, tgt_boxes)

    body_b = functools.partial(_assign_kernel, bs, nq, nt, npad)
    match_p = pl.pallas_call(
        body_b,
        out_shape=jax.ShapeDtypeStruct((bs, R, 128), jnp.int32),
    )(W, tgt_boxes, rpT_pad)

    match = match_p.reshape(bs, npad)[:, :nq].astype(tgt_labels.dtype)
    return (match, C)


# speedup vs baseline: 1.0175x; 1.0175x over previous
import functools

import jax
import jax.numpy as jnp
from jax import lax
from jax.experimental import pallas as pl
from jax.experimental.pallas import tpu as pltpu

ALPHA = 0.25
GAMMA = 2.0
COST_CLASS = 2.0
COST_BBOX = 5.0
COST_GIOU = 2.0
REF_DISTANCE = 0.1
INF = 1e18
BIG = 1 << 30


def _cost_kernel(nq, nt, ncls, npad, logits_ref, dx_ref, pb_ref, pbT_ref,
                 tlT_ref, ids_ref, tbT_ref, tbb_ref, c_ref, w_ref):
    R = npad // 128

    lg = logits_ref[0]
    prob = jax.nn.sigmoid(lg)
    neg = (1.0 - ALPHA) * (prob * prob) * -jnp.log(1.0 - prob + 1e-08)
    pos = ALPHA * ((1.0 - prob) * (1.0 - prob)) * -jnp.log(prob + 1e-08)
    diff = pos - neg

    ids = ids_ref[...]
    nall = ids.shape[1]
    e_full = (lax.broadcasted_iota(jnp.int32, (ncls, nall), 0) == ids
              ).astype(jnp.float32)
    cc = jnp.dot(diff, e_full, preferred_element_type=jnp.float32)

    ob = pb_ref[0]
    tbT = tbT_ref[...]
    cb = (jnp.abs(ob[:, 0:1] - tbT[0:1, :]) + jnp.abs(ob[:, 1:2] - tbT[1:2, :])
          + jnp.abs(ob[:, 2:3] - tbT[2:3, :]) + jnp.abs(ob[:, 3:4] - tbT[3:4, :]))

    ox0 = ob[:, 0:1] - 0.5 * ob[:, 2:3]
    oy0 = ob[:, 1:2] - 0.5 * ob[:, 3:4]
    ox1 = ob[:, 0:1] + 0.5 * ob[:, 2:3]
    oy1 = ob[:, 1:2] + 0.5 * ob[:, 3:4]
    tx0 = tbT[0:1, :] - 0.5 * tbT[2:3, :]
    ty0 = tbT[1:2, :] - 0.5 * tbT[3:4, :]
    tx1 = tbT[0:1, :] + 0.5 * tbT[2:3, :]
    ty1 = tbT[1:2, :] + 0.5 * tbT[3:4, :]
    area1 = (ox1 - ox0) * (oy1 - oy0)
    area2 = (tx1 - tx0) * (ty1 - ty0)
    iw = jnp.maximum(jnp.minimum(ox1, tx1) - jnp.maximum(ox0, tx0), 0.0)
    ih = jnp.maximum(jnp.minimum(oy1, ty1) - jnp.maximum(oy0, ty0), 0.0)
    inter = iw * ih
    union = area1 + area2 - inter
    iou = inter / union
    ew = jnp.maximum(jnp.maximum(ox1, tx1) - jnp.minimum(ox0, tx0), 0.0)
    eh = jnp.maximum(jnp.maximum(oy1, ty1) - jnp.minimum(oy0, ty0), 0.0)
    earea = ew * eh
    giou = iou - (earea - union) / earea
    c_ref[0] = COST_BBOX * cb + COST_CLASS * cc + COST_GIOU * (-giou)

    idsb = tlT_ref[0]
    eb = (lax.broadcasted_iota(jnp.int32, (nt, ncls), 1) == idsb
          ).astype(jnp.float32)
    ccT = lax.dot_general(eb, dx_ref[0], (((1,), (1,)), ((), ())),
                          preferred_element_type=jnp.float32,
                          precision=lax.Precision.HIGHEST)

    obT = pbT_ref[0]
    tbb = tbb_ref[0]
    cbT = (jnp.abs(tbb[:, 0:1] - obT[0:1, :]) + jnp.abs(tbb[:, 1:2] - obT[1:2, :])
           + jnp.abs(tbb[:, 2:3] - obT[2:3, :]) + jnp.abs(tbb[:, 3:4] - obT[3:4, :]))

    qx0 = obT[0:1, :] - 0.5 * obT[2:3, :]
    qy0 = obT[1:2, :] - 0.5 * obT[3:4, :]
    qx1 = obT[0:1, :] + 0.5 * obT[2:3, :]
    qy1 = obT[1:2, :] + 0.5 * obT[3:4, :]
    gx0 = tbb[:, 0:1] - 0.5 * tbb[:, 2:3]
    gy0 = tbb[:, 1:2] - 0.5 * tbb[:, 3:4]
    gx1 = tbb[:, 0:1] + 0.5 * tbb[:, 2:3]
    gy1 = tbb[:, 1:2] + 0.5 * tbb[:, 3:4]
    areaq = (qx1 - qx0) * (qy1 - qy0)
    areag = (gx1 - gx0) * (gy1 - gy0)
    iwT = jnp.maximum(jnp.minimum(qx1, gx1) - jnp.maximum(qx0, gx0), 0.0)
    ihT = jnp.maximum(jnp.minimum(qy1, gy1) - jnp.maximum(qy0, gy0), 0.0)
    interT = iwT * ihT
    unionT = areaq + areag - interT
    iouT = interT / unionT
    ewT = jnp.maximum(jnp.maximum(qx1, gx1) - jnp.minimum(qx0, gx0), 0.0)
    ehT = jnp.maximum(jnp.maximum(qy1, gy1) - jnp.minimum(qy0, gy0), 0.0)
    eareaT = ewT * ehT
    giouT = iouT - (eareaT - unionT) / eareaT
    subT = COST_BBOX * cbT + COST_CLASS * ccT + COST_GIOU * (-giouT)

    subT_pad = jnp.concatenate(
        [subT, jnp.zeros((nt, npad - nq), jnp.float32)], axis=1)
    for k in range(R):
        w_ref[:, 0, k, :] = subT_pad[:, k * 128:(k + 1) * 128]


def _assign_kernel(bs, nq, nt, npad, w_ref, tbb_ref, rpT_ref, match_ref):
    R = npad // 128
    nrow_pad = 32

    fi3 = (lax.broadcasted_iota(jnp.int32, (1, R, 128), 1) * 128
           + lax.broadcasted_iota(jnp.int32, (1, R, 128), 2))
    pad_used = jnp.broadcast_to((fi3 >= nq).astype(jnp.float32), (bs, R, 128))
    riota = lax.broadcasted_iota(jnp.int32, (1, nrow_pad), 1)

    u0 = jnp.zeros((bs, nrow_pad), jnp.float32)
    v0 = jnp.zeros((bs, R, 128), jnp.float32)
    p0 = jnp.full((bs, R, 128), -1, jnp.int32)
    way0 = jnp.full((bs, R, 128), -1, jnp.int32)

    def row_body(i, carry):
        u, v, p, way = carry
        packed = fi3 * 32 + (p + 1)

        def s_cond(st):
            return jnp.max(st[0]) > 0

        def s_body(st):
            act, r, j0, minv, usedc, usedr, way, u, v = st
            rows = []
            for im in range(bs):
                rows.append(w_ref[r[im, 0, 0], im])
            rows = jnp.stack(rows, axis=0)
            actb = act > 0
            act8 = actb[:, :, 0]
            ucur = jnp.sum(u * (riota == r[:, :, 0]).astype(jnp.float32),
                           axis=1, keepdims=True)
            cur = rows - ucur[:, :, None] - v
            freeb = usedc < 0.5
            better = actb & freeb & (cur < minv)
            minv = jnp.where(better, cur, minv)
            way = jnp.where(better, j0, way)
            masked = jnp.where(freeb, minv, INF)
            delta = jnp.min(jnp.min(masked, axis=1, keepdims=True),
                            axis=2, keepdims=True)
            key = jnp.where(actb & (masked == delta), packed, jnp.int32(BIG))
            kmin = jnp.min(jnp.min(key, axis=1, keepdims=True),
                           axis=2, keepdims=True)
            j1 = kmin // 32
            rnew = (kmin & 31) - 1
            adelta = jnp.where(act8, delta[:, :, 0], 0.0)
            u = u + adelta * usedr
            v = v - adelta[:, :, None] * usedc
            minv = jnp.where(freeb, minv - adelta[:, :, None], minv)
            selj = fi3 == j1
            usedc = jnp.where(selj, 1.0, usedc)
            usedr = jnp.where((riota == rnew[:, :, 0]) & act8, 1.0, usedr)
            j0 = jnp.where(actb, j1, j0)
            act = jnp.where(actb & (rnew >= 0), 1, 0)
            rnext = jnp.maximum(rnew, 0)
            return (act, rnext, j0, minv, usedc, usedr, way, u, v)

        st0 = (jnp.ones((bs, 1, 1), jnp.int32),
               jnp.full((bs, 1, 1), i, jnp.int32),
               jnp.full((bs, 1, 1), -1, jnp.int32),
               jnp.full((bs, R, 128), INF, jnp.float32),
               pad_used,
               jnp.broadcast_to((riota == i).astype(jnp.float32),
                                (bs, nrow_pad)),
               way, u, v)
        (_, _, jfin, _, _, _, way, u, v) = lax.while_loop(
            s_cond, s_body, st0)

        def a_cond(st):
            return jnp.max(st[0]) > 0

        def a_body(st):
            act, j, p = st
            actb = act > 0
            selj = actb & (fi3 == j)
            pj = jnp.min(jnp.min(jnp.where(selj, way, jnp.int32(BIG)),
                                 axis=1, keepdims=True),
                         axis=2, keepdims=True)
            pj = jnp.where(actb, pj, jnp.int32(-2))
            pval = jnp.sum(jnp.sum(jnp.where(fi3 == pj, p, 0),
                                   axis=1, keepdims=True),
                           axis=2, keepdims=True)
            newp = jnp.where(pj < 0, i, pval)
            p = jnp.where(selj, newp, p)
            act = jnp.where(actb & (pj >= 0), 1, 0)
            return (act, pj, p)

        _, _, p = lax.while_loop(
            a_cond, a_body, (jnp.ones((bs, 1, 1), jnp.int32), jfin, p))
        return (u, v, p, way)

    _, _, p, _ = lax.fori_loop(0, nt, row_body, (u0, v0, p0, way0))

    tbb = tbb_ref[...]
    gcx = tbb[:, :, 0:1]
    gcy = tbb[:, :, 1:2]
    tiota = lax.broadcasted_iota(jnp.int32, (bs, nt, 128), 1)
    mind_rows = []
    midx_rows = []
    for k in range(R):
        rx = rpT_ref[:, 0:1, k * 128:(k + 1) * 128]
        ry = rpT_ref[:, 1:2, k * 128:(k + 1) * 128]
        dxk = rx - gcx
        dyk = ry - gcy
        dk = jnp.sqrt(dxk * dxk + dyk * dyk)
        mk = jnp.min(dk, axis=1, keepdims=True)
        ik = jnp.min(jnp.where(dk == mk, tiota, jnp.int32(BIG)), axis=1, keepdims=True)
        mind_rows.append(mk)
        midx_rows.append(ik)
    min_d = jnp.concatenate(mind_rows, axis=1)
    min_idx = jnp.concatenate(midx_rows, axis=1)
    sel = (p < 0) & (min_d <= REF_DISTANCE)
    match_ref[...] = jnp.where(sel, min_idx, p)


def kernel(pred_logits, pred_boxes, tgt_labels, tgt_boxes, reference_points):
    bs, nq, ncls = pred_logits.shape
    nt = tgt_boxes.shape[1]
    npad = ((nq + 1023) // 1024) * 1024
    R = npad // 128

    out_prob = jax.nn.sigmoid(pred_logits.reshape(bs * nq, ncls))
    neg_c = (1 - ALPHA) * out_prob ** GAMMA * -jnp.log(1 - out_prob + 1e-08)
    pos_c = ALPHA * (1 - out_prob) ** GAMMA * -jnp.log(out_prob + 1e-08)
    dx = (pos_c - neg_c).reshape(bs, nq, ncls)

    ids_full = tgt_labels.reshape(1, bs * nt).astype(jnp.int32)
    tlT = tgt_labels.astype(jnp.int32).reshape(bs, nt, 1)
    tbT_full = tgt_boxes.reshape(bs * nt, 4).T
    pbT = jnp.swapaxes(pred_boxes, 1, 2)
    rpT = jnp.swapaxes(reference_points, 1, 2)
    rpT_pad = jnp.pad(rpT, ((0, 0), (0, 0), (0, npad - nq)),
                      constant_values=1e6)

    body_a = functools.partial(_cost_kernel, nq, nt, ncls, npad)
    C, W = pl.pallas_call(
        body_a,
        grid=(bs,),
        in_specs=[
            pl.BlockSpec((1, nq, ncls), lambda b: (b, 0, 0)),
            pl.BlockSpec((1, nq, ncls), lambda b: (b, 0, 0)),
            pl.BlockSpec((1, nq, 4), lambda b: (b, 0, 0)),
            pl.BlockSpec((1, 4, nq), lambda b: (b, 0, 0)),
            pl.BlockSpec((1, nt, 1), lambda b: (b, 0, 0)),
            pl.BlockSpec((1, bs * nt), lambda b: (0, 0)),
            pl.BlockSpec((4, bs * nt), lambda b: (0, 0)),
            pl.BlockSpec((1, nt, 4), lambda b: (b, 0, 0)),
        ],
        out_specs=[
            pl.BlockSpec((1, nq, bs * nt), lambda b: (b, 0, 0)),
            pl.BlockSpec((nt, 1, R, 128), lambda b: (0, b, 0, 0)),
        ],
        out_shape=[
            jax.ShapeDtypeStruct((bs, nq, bs * nt), jnp.float32),
            jax.ShapeDtypeStruct((nt, bs, R, 128), jnp.float32),
        ],
        compiler_params=pltpu.CompilerParams(
            dimension_semantics=("arbitrary",)),
    )(pred_logits, dx, pred_boxes, pbT, tlT, ids_full, tbT_full, tgt_boxes)

    body_b = functools.partial(_assign_kernel, bs, nq, nt, npad)
    match_p = pl.pallas_call(
        body_b,
        out_shape=jax.ShapeDtypeStruct((bs, R, 128), jnp.int32),
    )(W, tgt_boxes, rpT_pad)

    match = match_p.reshape(bs, npad)[:, :nq].astype(tgt_labels.dtype)
    return (match, C)


# speedup vs baseline: 1.0830x; 1.0644x over previous
import functools

import jax
import jax.numpy as jnp
from jax import lax
from jax.experimental import pallas as pl
from jax.experimental.pallas import tpu as pltpu

ALPHA = 0.25
GAMMA = 2.0
COST_CLASS = 2.0
COST_BBOX = 5.0
COST_GIOU = 2.0
REF_DISTANCE = 0.1
INF = 1e18
BIG = 1 << 30


def _cost_kernel(nq, nt, ncls, npad, logits_ref, dx_ref, pb_ref, pbT_ref,
                 tlT_ref, ids_ref, tbT_ref, tbb_ref, c_ref, w_ref):
    R = npad // 128

    lg = logits_ref[0]
    prob = jax.nn.sigmoid(lg)
    neg = (1.0 - ALPHA) * (prob * prob) * -jnp.log(1.0 - prob + 1e-08)
    pos = ALPHA * ((1.0 - prob) * (1.0 - prob)) * -jnp.log(prob + 1e-08)
    diff = pos - neg

    ids = ids_ref[...]
    nall = ids.shape[1]
    e_full = (lax.broadcasted_iota(jnp.int32, (ncls, nall), 0) == ids
              ).astype(jnp.float32)
    cc = jnp.dot(diff, e_full, preferred_element_type=jnp.float32)

    ob = pb_ref[0]
    tbT = tbT_ref[...]
    cb = (jnp.abs(ob[:, 0:1] - tbT[0:1, :]) + jnp.abs(ob[:, 1:2] - tbT[1:2, :])
          + jnp.abs(ob[:, 2:3] - tbT[2:3, :]) + jnp.abs(ob[:, 3:4] - tbT[3:4, :]))

    ox0 = ob[:, 0:1] - 0.5 * ob[:, 2:3]
    oy0 = ob[:, 1:2] - 0.5 * ob[:, 3:4]
    ox1 = ob[:, 0:1] + 0.5 * ob[:, 2:3]
    oy1 = ob[:, 1:2] + 0.5 * ob[:, 3:4]
    tx0 = tbT[0:1, :] - 0.5 * tbT[2:3, :]
    ty0 = tbT[1:2, :] - 0.5 * tbT[3:4, :]
    tx1 = tbT[0:1, :] + 0.5 * tbT[2:3, :]
    ty1 = tbT[1:2, :] + 0.5 * tbT[3:4, :]
    area1 = (ox1 - ox0) * (oy1 - oy0)
    area2 = (tx1 - tx0) * (ty1 - ty0)
    iw = jnp.maximum(jnp.minimum(ox1, tx1) - jnp.maximum(ox0, tx0), 0.0)
    ih = jnp.maximum(jnp.minimum(oy1, ty1) - jnp.maximum(oy0, ty0), 0.0)
    inter = iw * ih
    union = area1 + area2 - inter
    iou = inter / union
    ew = jnp.maximum(jnp.maximum(ox1, tx1) - jnp.minimum(ox0, tx0), 0.0)
    eh = jnp.maximum(jnp.maximum(oy1, ty1) - jnp.minimum(oy0, ty0), 0.0)
    earea = ew * eh
    giou = iou - (earea - union) / earea
    c_ref[0] = COST_BBOX * cb + COST_CLASS * cc + COST_GIOU * (-giou)

    idsb = tlT_ref[0]
    eb = (lax.broadcasted_iota(jnp.int32, (nt, ncls), 1) == idsb
          ).astype(jnp.float32)
    ccT = lax.dot_general(eb, dx_ref[0], (((1,), (1,)), ((), ())),
                          preferred_element_type=jnp.float32,
                          precision=lax.Precision.HIGHEST)

    obT = pbT_ref[0]
    tbb = tbb_ref[0]
    cbT = (jnp.abs(tbb[:, 0:1] - obT[0:1, :]) + jnp.abs(tbb[:, 1:2] - obT[1:2, :])
           + jnp.abs(tbb[:, 2:3] - obT[2:3, :]) + jnp.abs(tbb[:, 3:4] - obT[3:4, :]))

    qx0 = obT[0:1, :] - 0.5 * obT[2:3, :]
    qy0 = obT[1:2, :] - 0.5 * obT[3:4, :]
    qx1 = obT[0:1, :] + 0.5 * obT[2:3, :]
    qy1 = obT[1:2, :] + 0.5 * obT[3:4, :]
    gx0 = tbb[:, 0:1] - 0.5 * tbb[:, 2:3]
    gy0 = tbb[:, 1:2] - 0.5 * tbb[:, 3:4]
    gx1 = tbb[:, 0:1] + 0.5 * tbb[:, 2:3]
    gy1 = tbb[:, 1:2] + 0.5 * tbb[:, 3:4]
    areaq = (qx1 - qx0) * (qy1 - qy0)
    areag = (gx1 - gx0) * (gy1 - gy0)
    iwT = jnp.maximum(jnp.minimum(qx1, gx1) - jnp.maximum(qx0, gx0), 0.0)
    ihT = jnp.maximum(jnp.minimum(qy1, gy1) - jnp.maximum(qy0, gy0), 0.0)
    interT = iwT * ihT
    unionT = areaq + areag - interT
    iouT = interT / unionT
    ewT = jnp.maximum(jnp.maximum(qx1, gx1) - jnp.minimum(qx0, gx0), 0.0)
    ehT = jnp.maximum(jnp.maximum(qy1, gy1) - jnp.minimum(qy0, gy0), 0.0)
    eareaT = ewT * ehT
    giouT = iouT - (eareaT - unionT) / eareaT
    subT = COST_BBOX * cbT + COST_CLASS * ccT + COST_GIOU * (-giouT)

    subT_pad = jnp.concatenate(
        [subT, jnp.zeros((nt, npad - nq), jnp.float32)], axis=1)
    for k in range(R):
        w_ref[0, :, k, :] = subT_pad[:, k * 128:(k + 1) * 128]


def _assign_kernel(bs, nq, nt, npad, w_ref, tbb_ref, rpT_ref, match_ref):
    R = npad // 128
    nrow_pad = 32

    fi3 = (lax.broadcasted_iota(jnp.int32, (1, R, 128), 1) * 128
           + lax.broadcasted_iota(jnp.int32, (1, R, 128), 2))
    pad_used = jnp.broadcast_to((fi3 >= nq).astype(jnp.float32), (bs, R, 128))
    riota = lax.broadcasted_iota(jnp.int32, (1, nrow_pad), 1)

    u0 = jnp.zeros((bs, nrow_pad), jnp.float32)
    v0 = jnp.zeros((bs, R, 128), jnp.float32)
    p0 = jnp.full((bs, R, 128), -1, jnp.int32)
    way0 = jnp.full((bs, R, 128), -1, jnp.int32)

    def row_body(i, carry):
        u, v, p, way = carry
        packed = fi3 * 32 + (p + 1)

        def s_cond(st):
            return jnp.max(st[0]) > 0

        def s_body(st):
            act, r, j0, minv, usedc, usedr, way, u, v = st
            rows = []
            for im in range(bs):
                rows.append(w_ref[im, r[im, 0, 0]])
            rows = jnp.stack(rows, axis=0)
            actb = act > 0
            act8 = actb[:, :, 0]
            ucur = jnp.sum(u * (riota == r[:, :, 0]).astype(jnp.float32),
                           axis=1, keepdims=True)
            cur = rows - ucur[:, :, None] - v
            freeb = usedc < 0.5
            better = actb & freeb & (cur < minv)
            minv = jnp.where(better, cur, minv)
            way = jnp.where(better, j0, way)
            masked = jnp.where(freeb, minv, INF)
            delta = jnp.min(jnp.min(masked, axis=1, keepdims=True),
                            axis=2, keepdims=True)
            key = jnp.where(actb & (masked == delta), packed, jnp.int32(BIG))
            kmin = jnp.min(jnp.min(key, axis=1, keepdims=True),
                           axis=2, keepdims=True)
            j1 = kmin // 32
            rnew = (kmin & 31) - 1
            adelta = jnp.where(act8, delta[:, :, 0], 0.0)
            u = u + adelta * usedr
            v = v - adelta[:, :, None] * usedc
            minv = jnp.where(freeb, minv - adelta[:, :, None], minv)
            selj = fi3 == j1
            usedc = jnp.where(selj, 1.0, usedc)
            usedr = jnp.where((riota == rnew[:, :, 0]) & act8, 1.0, usedr)
            j0 = jnp.where(actb, j1, j0)
            act = jnp.where(actb & (rnew >= 0), 1, 0)
            rnext = jnp.maximum(rnew, 0)
            return (act, rnext, j0, minv, usedc, usedr, way, u, v)

        st0 = (jnp.ones((bs, 1, 1), jnp.int32),
               jnp.full((bs, 1, 1), i, jnp.int32),
               jnp.full((bs, 1, 1), -1, jnp.int32),
               jnp.full((bs, R, 128), INF, jnp.float32),
               pad_used,
               jnp.broadcast_to((riota == i).astype(jnp.float32),
                                (bs, nrow_pad)),
               way, u, v)
        (_, _, jfin, _, _, _, way, u, v) = lax.while_loop(
            s_cond, s_body, st0)

        def a_cond(st):
            return jnp.max(st[0]) > 0

        def a_body(st):
            act, j, p = st
            actb = act > 0
            selj = actb & (fi3 == j)
            pj = jnp.min(jnp.min(jnp.where(selj, way, jnp.int32(BIG)),
                                 axis=1, keepdims=True),
                         axis=2, keepdims=True)
            pj = jnp.where(actb, pj, jnp.int32(-2))
            pval = jnp.sum(jnp.sum(jnp.where(fi3 == pj, p, 0),
                                   axis=1, keepdims=True),
                           axis=2, keepdims=True)
            newp = jnp.where(pj < 0, i, pval)
            p = jnp.where(selj, newp, p)
            act = jnp.where(actb & (pj >= 0), 1, 0)
            return (act, pj, p)

        _, _, p = lax.while_loop(
            a_cond, a_body, (jnp.ones((bs, 1, 1), jnp.int32), jfin, p))
        return (u, v, p, way)

    _, _, p, _ = lax.fori_loop(0, nt, row_body, (u0, v0, p0, way0))

    tbb = tbb_ref[...]
    gcx = tbb[:, :, 0:1]
    gcy = tbb[:, :, 1:2]
    tiota = lax.broadcasted_iota(jnp.int32, (bs, nt, 128), 1)
    mind_rows = []
    midx_rows = []
    for k in range(R):
        rx = rpT_ref[:, 0:1, k * 128:(k + 1) * 128]
        ry = rpT_ref[:, 1:2, k * 128:(k + 1) * 128]
        dxk = rx - gcx
        dyk = ry - gcy
        dk = jnp.sqrt(dxk * dxk + dyk * dyk)
        mk = jnp.min(dk, axis=1, keepdims=True)
        ik = jnp.min(jnp.where(dk == mk, tiota, jnp.int32(BIG)), axis=1, keepdims=True)
        mind_rows.append(mk)
        midx_rows.append(ik)
    min_d = jnp.concatenate(mind_rows, axis=1)
    min_idx = jnp.concatenate(midx_rows, axis=1)
    sel = (p < 0) & (min_d <= REF_DISTANCE)
    match_ref[...] = jnp.where(sel, min_idx, p)


def kernel(pred_logits, pred_boxes, tgt_labels, tgt_boxes, reference_points):
    bs, nq, ncls = pred_logits.shape
    nt = tgt_boxes.shape[1]
    npad = ((nq + 1023) // 1024) * 1024
    R = npad // 128

    out_prob = jax.nn.sigmoid(pred_logits.reshape(bs * nq, ncls))
    neg_c = (1 - ALPHA) * out_prob ** GAMMA * -jnp.log(1 - out_prob + 1e-08)
    pos_c = ALPHA * (1 - out_prob) ** GAMMA * -jnp.log(out_prob + 1e-08)
    dx = (pos_c - neg_c).reshape(bs, nq, ncls)

    ids_full = tgt_labels.reshape(1, bs * nt).astype(jnp.int32)
    tlT = tgt_labels.astype(jnp.int32).reshape(bs, nt, 1)
    tbT_full = tgt_boxes.reshape(bs * nt, 4).T
    pbT = jnp.swapaxes(pred_boxes, 1, 2)
    rpT = jnp.swapaxes(reference_points, 1, 2)
    rpT_pad = jnp.pad(rpT, ((0, 0), (0, 0), (0, npad - nq)),
                      constant_values=1e6)

    body_a = functools.partial(_cost_kernel, nq, nt, ncls, npad)
    C, W = pl.pallas_call(
        body_a,
        grid=(bs,),
        in_specs=[
            pl.BlockSpec((1, nq, ncls), lambda b: (b, 0, 0)),
            pl.BlockSpec((1, nq, ncls), lambda b: (b, 0, 0)),
            pl.BlockSpec((1, nq, 4), lambda b: (b, 0, 0)),
            pl.BlockSpec((1, 4, nq), lambda b: (b, 0, 0)),
            pl.BlockSpec((1, nt, 1), lambda b: (b, 0, 0)),
            pl.BlockSpec((1, bs * nt), lambda b: (0, 0)),
            pl.BlockSpec((4, bs * nt), lambda b: (0, 0)),
            pl.BlockSpec((1, nt, 4), lambda b: (b, 0, 0)),
        ],
        out_specs=[
            pl.BlockSpec((1, nq, bs * nt), lambda b: (b, 0, 0)),
            pl.BlockSpec((1, nt, R, 128), lambda b: (b, 0, 0, 0)),
        ],
        out_shape=[
            jax.ShapeDtypeStruct((bs, nq, bs * nt), jnp.float32),
            jax.ShapeDtypeStruct((bs, nt, R, 128), jnp.float32),
        ],
        compiler_params=pltpu.CompilerParams(
            dimension_semantics=("arbitrary",)),
    )(pred_logits, dx, pred_boxes, pbT, tlT, ids_full, tbT_full, tgt_boxes)

    body_b = functools.partial(_assign_kernel, bs, nq, nt, npad)
    match_p = pl.pallas_call(
        body_b,
        out_shape=jax.ShapeDtypeStruct((bs, R, 128), jnp.int32),
    )(W, tgt_boxes, rpT_pad)

    match = match_p.reshape(bs, npad)[:, :nq].astype(tgt_labels.dtype)
    return (match, C)


# speedup vs baseline: 1.0857x; 1.0025x over previous
import functools

import jax
import jax.numpy as jnp
from jax import lax
from jax.experimental import pallas as pl
from jax.experimental.pallas import tpu as pltpu

ALPHA = 0.25
GAMMA = 2.0
COST_CLASS = 2.0
COST_BBOX = 5.0
COST_GIOU = 2.0
REF_DISTANCE = 0.1
INF = 1e18
BIG = 1 << 30


def _cost_kernel(nq, nt, ncls, npad, logits_ref, dx_ref, pb_ref, pbT_ref,
                 tlT_ref, ids_ref, tbT_ref, tbb_ref, c_ref, w_ref):
    R = npad // 128

    lg = logits_ref[0]
    prob = jax.nn.sigmoid(lg)
    neg = (1.0 - ALPHA) * (prob * prob) * -jnp.log(1.0 - prob + 1e-08)
    pos = ALPHA * ((1.0 - prob) * (1.0 - prob)) * -jnp.log(prob + 1e-08)
    diff = pos - neg

    ids = ids_ref[...]
    nall = ids.shape[1]
    e_full = (lax.broadcasted_iota(jnp.int32, (ncls, nall), 0) == ids
              ).astype(jnp.float32)
    cc = jnp.dot(diff, e_full, preferred_element_type=jnp.float32)

    ob = pb_ref[0]
    tbT = tbT_ref[...]
    cb = (jnp.abs(ob[:, 0:1] - tbT[0:1, :]) + jnp.abs(ob[:, 1:2] - tbT[1:2, :])
          + jnp.abs(ob[:, 2:3] - tbT[2:3, :]) + jnp.abs(ob[:, 3:4] - tbT[3:4, :]))

    ox0 = ob[:, 0:1] - 0.5 * ob[:, 2:3]
    oy0 = ob[:, 1:2] - 0.5 * ob[:, 3:4]
    ox1 = ob[:, 0:1] + 0.5 * ob[:, 2:3]
    oy1 = ob[:, 1:2] + 0.5 * ob[:, 3:4]
    tx0 = tbT[0:1, :] - 0.5 * tbT[2:3, :]
    ty0 = tbT[1:2, :] - 0.5 * tbT[3:4, :]
    tx1 = tbT[0:1, :] + 0.5 * tbT[2:3, :]
    ty1 = tbT[1:2, :] + 0.5 * tbT[3:4, :]
    area1 = (ox1 - ox0) * (oy1 - oy0)
    area2 = (tx1 - tx0) * (ty1 - ty0)
    iw = jnp.maximum(jnp.minimum(ox1, tx1) - jnp.maximum(ox0, tx0), 0.0)
    ih = jnp.maximum(jnp.minimum(oy1, ty1) - jnp.maximum(oy0, ty0), 0.0)
    inter = iw * ih
    union = area1 + area2 - inter
    iou = inter / union
    ew = jnp.maximum(jnp.maximum(ox1, tx1) - jnp.minimum(ox0, tx0), 0.0)
    eh = jnp.maximum(jnp.maximum(oy1, ty1) - jnp.minimum(oy0, ty0), 0.0)
    earea = ew * eh
    giou = iou - (earea - union) / earea
    c_ref[0] = COST_BBOX * cb + COST_CLASS * cc + COST_GIOU * (-giou)

    idsb = tlT_ref[0]
    eb = (lax.broadcasted_iota(jnp.int32, (nt, ncls), 1) == idsb
          ).astype(jnp.float32)
    ccT = lax.dot_general(eb, dx_ref[0], (((1,), (1,)), ((), ())),
                          preferred_element_type=jnp.float32,
                          precision=lax.Precision.HIGHEST)

    obT = pbT_ref[0]
    tbb = tbb_ref[0]
    cbT = (jnp.abs(tbb[:, 0:1] - obT[0:1, :]) + jnp.abs(tbb[:, 1:2] - obT[1:2, :])
           + jnp.abs(tbb[:, 2:3] - obT[2:3, :]) + jnp.abs(tbb[:, 3:4] - obT[3:4, :]))

    qx0 = obT[0:1, :] - 0.5 * obT[2:3, :]
    qy0 = obT[1:2, :] - 0.5 * obT[3:4, :]
    qx1 = obT[0:1, :] + 0.5 * obT[2:3, :]
    qy1 = obT[1:2, :] + 0.5 * obT[3:4, :]
    gx0 = tbb[:, 0:1] - 0.5 * tbb[:, 2:3]
    gy0 = tbb[:, 1:2] - 0.5 * tbb[:, 3:4]
    gx1 = tbb[:, 0:1] + 0.5 * tbb[:, 2:3]
    gy1 = tbb[:, 1:2] + 0.5 * tbb[:, 3:4]
    areaq = (qx1 - qx0) * (qy1 - qy0)
    areag = (gx1 - gx0) * (gy1 - gy0)
    iwT = jnp.maximum(jnp.minimum(qx1, gx1) - jnp.maximum(qx0, gx0), 0.0)
    ihT = jnp.maximum(jnp.minimum(qy1, gy1) - jnp.maximum(qy0, gy0), 0.0)
    interT = iwT * ihT
    unionT = areaq + areag - interT
    iouT = interT / unionT
    ewT = jnp.maximum(jnp.maximum(qx1, gx1) - jnp.minimum(qx0, gx0), 0.0)
    ehT = jnp.maximum(jnp.maximum(qy1, gy1) - jnp.minimum(qy0, gy0), 0.0)
    eareaT = ewT * ehT
    giouT = iouT - (eareaT - unionT) / eareaT
    subT = COST_BBOX * cbT + COST_CLASS * ccT + COST_GIOU * (-giouT)

    subT_pad = jnp.concatenate(
        [subT, jnp.zeros((nt, npad - nq), jnp.float32)], axis=1)
    for k in range(R):
        w_ref[0, :, k, :] = subT_pad[:, k * 128:(k + 1) * 128]


def _assign_kernel(bs, nq, nt, npad, w_ref, tbb_ref, rpT_ref, match_ref):
    R = npad // 128
    RS = (nq + 127) // 128
    nrow_pad = 32

    fi3 = (lax.broadcasted_iota(jnp.int32, (1, RS, 128), 1) * 128
           + lax.broadcasted_iota(jnp.int32, (1, RS, 128), 2))
    pad_used = jnp.broadcast_to((fi3 >= nq).astype(jnp.float32), (bs, RS, 128))
    riota = lax.broadcasted_iota(jnp.int32, (1, nrow_pad), 1)

    u0 = jnp.zeros((bs, nrow_pad), jnp.float32)
    v0 = jnp.zeros((bs, RS, 128), jnp.float32)
    p0 = jnp.full((bs, RS, 128), -1, jnp.int32)
    way0 = jnp.full((bs, RS, 128), -1, jnp.int32)

    def row_body(i, carry):
        u, v, p, way = carry
        packed = fi3 * 32 + (p + 1)

        def s_cond(st):
            return jnp.max(st[0]) > 0

        def s_body(st):
            act, r, j0, minv, usedc, usedr, way, u, v = st
            rows = []
            for im in range(bs):
                rows.append(w_ref[im, r[im, 0, 0]][0:RS])
            rows = jnp.stack(rows, axis=0)
            actb = act > 0
            act8 = actb[:, :, 0]
            ucur = jnp.sum(u * (riota == r[:, :, 0]).astype(jnp.float32),
                           axis=1, keepdims=True)
            cur = rows - ucur[:, :, None] - v
            freeb = usedc < 0.5
            better = actb & freeb & (cur < minv)
            minv = jnp.where(better, cur, minv)
            way = jnp.where(better, j0, way)
            masked = jnp.where(freeb, minv, INF)
            delta = jnp.min(jnp.min(masked, axis=1, keepdims=True),
                            axis=2, keepdims=True)
            key = jnp.where(actb & (masked == delta), packed, jnp.int32(BIG))
            kmin = jnp.min(jnp.min(key, axis=1, keepdims=True),
                           axis=2, keepdims=True)
            j1 = kmin // 32
            rnew = (kmin & 31) - 1
            adelta = jnp.where(act8, delta[:, :, 0], 0.0)
            u = u + adelta * usedr
            v = v - adelta[:, :, None] * usedc
            minv = jnp.where(freeb, minv - adelta[:, :, None], minv)
            selj = fi3 == j1
            usedc = jnp.where(selj, 1.0, usedc)
            usedr = jnp.where((riota == rnew[:, :, 0]) & act8, 1.0, usedr)
            j0 = jnp.where(actb, j1, j0)
            act = jnp.where(actb & (rnew >= 0), 1, 0)
            rnext = jnp.maximum(rnew, 0)
            return (act, rnext, j0, minv, usedc, usedr, way, u, v)

        st0 = (jnp.ones((bs, 1, 1), jnp.int32),
               jnp.full((bs, 1, 1), i, jnp.int32),
               jnp.full((bs, 1, 1), -1, jnp.int32),
               jnp.full((bs, RS, 128), INF, jnp.float32),
               pad_used,
               jnp.broadcast_to((riota == i).astype(jnp.float32),
                                (bs, nrow_pad)),
               way, u, v)
        (_, _, jfin, _, _, _, way, u, v) = lax.while_loop(
            s_cond, s_body, st0)

        def a_cond(st):
            return jnp.max(st[0]) > 0

        def a_body(st):
            act, j, p = st
            actb = act > 0
            selj = actb & (fi3 == j)
            pj = jnp.min(jnp.min(jnp.where(selj, way, jnp.int32(BIG)),
                                 axis=1, keepdims=True),
                         axis=2, keepdims=True)
            pj = jnp.where(actb, pj, jnp.int32(-2))
            pval = jnp.sum(jnp.sum(jnp.where(fi3 == pj, p, 0),
                                   axis=1, keepdims=True),
                           axis=2, keepdims=True)
            newp = jnp.where(pj < 0, i, pval)
            p = jnp.where(selj, newp, p)
            act = jnp.where(actb & (pj >= 0), 1, 0)
            return (act, pj, p)

        _, _, p = lax.while_loop(
            a_cond, a_body, (jnp.ones((bs, 1, 1), jnp.int32), jfin, p))
        return (u, v, p, way)

    _, _, p, _ = lax.fori_loop(0, nt, row_body, (u0, v0, p0, way0))

    tbb = tbb_ref[...]
    gcx = tbb[:, :, 0:1]
    gcy = tbb[:, :, 1:2]
    tiota = lax.broadcasted_iota(jnp.int32, (bs, nt, 128), 1)
    mind_rows = []
    midx_rows = []
    for k in range(RS):
        rx = rpT_ref[:, 0:1, k * 128:(k + 1) * 128]
        ry = rpT_ref[:, 1:2, k * 128:(k + 1) * 128]
        dxk = rx - gcx
        dyk = ry - gcy
        dk = jnp.sqrt(dxk * dxk + dyk * dyk)
        mk = jnp.min(dk, axis=1, keepdims=True)
        ik = jnp.min(jnp.where(dk == mk, tiota, jnp.int32(BIG)), axis=1, keepdims=True)
        mind_rows.append(mk)
        midx_rows.append(ik)
    min_d = jnp.concatenate(mind_rows, axis=1)
    min_idx = jnp.concatenate(midx_rows, axis=1)
    sel = (p < 0) & (min_d <= REF_DISTANCE)
    matchv = jnp.where(sel, min_idx, p)
    match_ref[...] = jnp.concatenate(
        [matchv, jnp.full((bs, R - RS, 128), -1, jnp.int32)], axis=1)


def kernel(pred_logits, pred_boxes, tgt_labels, tgt_boxes, reference_points):
    bs, nq, ncls = pred_logits.shape
    nt = tgt_boxes.shape[1]
    npad = ((nq + 1023) // 1024) * 1024
    R = npad // 128

    out_prob = jax.nn.sigmoid(pred_logits.reshape(bs * nq, ncls))
    neg_c = (1 - ALPHA) * out_prob ** GAMMA * -jnp.log(1 - out_prob + 1e-08)
    pos_c = ALPHA * (1 - out_prob) ** GAMMA * -jnp.log(out_prob + 1e-08)
    dx = (pos_c - neg_c).reshape(bs, nq, ncls)

    ids_full = tgt_labels.reshape(1, bs * nt).astype(jnp.int32)
    tlT = tgt_labels.astype(jnp.int32).reshape(bs, nt, 1)
    tbT_full = tgt_boxes.reshape(bs * nt, 4).T
    pbT = jnp.swapaxes(pred_boxes, 1, 2)
    rpT = jnp.swapaxes(reference_points, 1, 2)
    rpT_pad = jnp.pad(rpT, ((0, 0), (0, 0), (0, npad - nq)),
                      constant_values=1e6)

    body_a = functools.partial(_cost_kernel, nq, nt, ncls, npad)
    C, W = pl.pallas_call(
        body_a,
        grid=(bs,),
        in_specs=[
            pl.BlockSpec((1, nq, ncls), lambda b: (b, 0, 0)),
            pl.BlockSpec((1, nq, ncls), lambda b: (b, 0, 0)),
            pl.BlockSpec((1, nq, 4), lambda b: (b, 0, 0)),
            pl.BlockSpec((1, 4, nq), lambda b: (b, 0, 0)),
            pl.BlockSpec((1, nt, 1), lambda b: (b, 0, 0)),
            pl.BlockSpec((1, bs * nt), lambda b: (0, 0)),
            pl.BlockSpec((4, bs * nt), lambda b: (0, 0)),
            pl.BlockSpec((1, nt, 4), lambda b: (b, 0, 0)),
        ],
        out_specs=[
            pl.BlockSpec((1, nq, bs * nt), lambda b: (b, 0, 0)),
            pl.BlockSpec((1, nt, R, 128), lambda b: (b, 0, 0, 0)),
        ],
        out_shape=[
            jax.ShapeDtypeStruct((bs, nq, bs * nt), jnp.float32),
            jax.ShapeDtypeStruct((bs, nt, R, 128), jnp.float32),
        ],
        compiler_params=pltpu.CompilerParams(
            dimension_semantics=("arbitrary",)),
    )(pred_logits, dx, pred_boxes, pbT, tlT, ids_full, tbT_full, tgt_boxes)

    body_b = functools.partial(_assign_kernel, bs, nq, nt, npad)
    match_p = pl.pallas_call(
        body_b,
        out_shape=jax.ShapeDtypeStruct((bs, R, 128), jnp.int32),
    )(W, tgt_boxes, rpT_pad)

    match = match_p.reshape(bs, npad)[:, :nq].astype(tgt_labels.dtype)
    return (match, C)
